# Initial kernel scaffold; baseline (speedup 1.0000x reference)
import jax
import jax.numpy as jnp
from jax.experimental import pallas as pl


def kernel(x):
    raise NotImplementedError("write your pallas kernel here")


# speedup vs baseline: 1.0001x; 1.0001x over previous
import numpy as np
import jax
import jax.numpy as jnp
from jax.experimental import pallas as pl
from jax.experimental.pallas import tpu as pltpu


def _pool_matrix(in_size: int, out_size: int) -> np.ndarray:
    A = np.zeros((out_size, in_size), dtype=np.float32)
    for i in range(out_size):
        s = (i * in_size) // out_size
        e = -((-(i + 1) * in_size) // out_size)
        A[i, s:e] = 1.0 / float(e - s)
    return A


def _matmul_kernel(x_ref, p_ref, o_ref):
    o_ref[...] = jnp.dot(
        x_ref[...], p_ref[...], preferred_element_type=jnp.float32
    ).astype(o_ref.dtype)


def _pool_call(x2d, p, row_tile):
    B, K = x2d.shape
    M = p.shape[1]
    try:
        w_spec = pl.BlockSpec((K, M), lambda b: (0, 0),
                              pipeline_mode=pl.Buffered(1))
    except Exception:
        w_spec = pl.BlockSpec((K, M), lambda b: (0, 0))
    return pl.pallas_call(
        _matmul_kernel,
        out_shape=jax.ShapeDtypeStruct((B, M), x2d.dtype),
        grid_spec=pl.GridSpec(
            grid=(pl.cdiv(B, row_tile),),
            in_specs=[pl.BlockSpec((row_tile, K), lambda b: (b, 0)), w_spec],
            out_specs=pl.BlockSpec((row_tile, M), lambda b: (b, 0)),
        ),
        compiler_params=pltpu.CompilerParams(
            dimension_semantics=("parallel",),
        ),
        cost_estimate=pl.CostEstimate(
            flops=2 * B * K * M,
            transcendentals=0,
            bytes_accessed=B * K * 4 + K * M * 4 + B * M * 4,
        ),
    )(x2d, p)


def kernel(x):
    N, C, H, W = x.shape
    OH, OW = 7, 7
    B, K = N * C, H * W
    p_t = jnp.asarray(
        np.kron(_pool_matrix(H, OH), _pool_matrix(W, OW)).T
    )
    out = _pool_call(x.reshape(B, K), p_t, row_tile=2048)
    return out.reshape(N, C, OH, OW)


# speedup vs baseline: 5.5065x; 5.5060x over previous
import jax
import jax.numpy as jnp
from jax.experimental import pallas as pl
from jax.experimental.pallas import tpu as pltpu


def _pool_kernel(x_ref, o_ref):
    s = x_ref[...].sum(axis=(0, 1))
    o_ref[0, 0] = s * (1.0 / (x_ref.shape[0] * x_ref.shape[1]))


def kernel(x):
    N, C, H, W = x.shape
    OH, OW = 7, 7
    assert H % OH == 0 and W % OW == 0
    sh, sw = H // OH, W // OW

    xt = x.transpose(2, 3, 0, 1)
    out_t = pl.pallas_call(
        _pool_kernel,
        out_shape=jax.ShapeDtypeStruct((OH, OW, N, C), x.dtype),
        grid_spec=pl.GridSpec(
            grid=(OH, OW),
            in_specs=[pl.BlockSpec((sh, sw, N, C), lambda i, j: (i, j, 0, 0))],
            out_specs=pl.BlockSpec((1, 1, N, C), lambda i, j: (i, j, 0, 0)),
        ),
        compiler_params=pltpu.CompilerParams(
            dimension_semantics=("parallel", "parallel"),
        ),
        cost_estimate=pl.CostEstimate(
            flops=N * C * H * W,
            transcendentals=0,
            bytes_accessed=4 * N * C * (H * W + OH * OW),
        ),
    )(xt)
    return out_t.transpose(2, 3, 0, 1)


# speedup vs baseline: 7.5066x; 1.3632x over previous
import functools

import jax
import jax.numpy as jnp
from jax.experimental import pallas as pl
from jax.experimental.pallas import tpu as pltpu


def _pool_kernel(ow, scale, x_ref, o_ref):
    hb, w, n, c = x_ref.shape
    t = x_ref[...].sum(axis=0).reshape(ow, w // ow, n, c).sum(axis=1) * scale
    s = pl.program_id(1)

    @pl.when(s == 0)
    def _init():
        o_ref[0] = t

    @pl.when(s != 0)
    def _acc():
        o_ref[0] += t


def kernel(x):
    N, C, H, W = x.shape
    OH, OW = 7, 7
    assert H % OH == 0 and W % OW == 0
    sh, sw = H // OH, W // OW
    hb = 2
    xt = x.transpose(2, 3, 0, 1)
    out_t = pl.pallas_call(
        functools.partial(_pool_kernel, OW, 1.0 / (sh * sw)),
        out_shape=jax.ShapeDtypeStruct((OH, OW, N, C), x.dtype),
        grid_spec=pl.GridSpec(
            grid=(OH, sh // hb),
            in_specs=[
                pl.BlockSpec(
                    (hb, W, N, C),
                    lambda i, s: (i * (sh // hb) + s, 0, 0, 0),
                )
            ],
            out_specs=pl.BlockSpec((1, OW, N, C), lambda i, s: (i, 0, 0, 0)),
        ),
        compiler_params=pltpu.CompilerParams(
            dimension_semantics=("parallel", "arbitrary"),
        ),
        cost_estimate=pl.CostEstimate(
            flops=N * C * H * W,
            transcendentals=0,
            bytes_accessed=4 * N * C * (H * W + OH * OW),
        ),
    )(xt)
    return out_t.transpose(2, 3, 0, 1)


# speedup vs baseline: 7.5945x; 1.0117x over previous
import functools

import jax
import jax.numpy as jnp
from jax.experimental import pallas as pl
from jax.experimental.pallas import tpu as pltpu


def _pool_kernel(ow, scale, x_ref, o_ref):
    sh, w, n, c = x_ref.shape
    o_ref[0] = (
        x_ref[...].sum(axis=0).reshape(ow, w // ow, n, c).sum(axis=1) * scale
    )


def kernel(x):
    N, C, H, W = x.shape
    OH, OW = 7, 7
    assert H % OH == 0 and W % OW == 0
    sh, sw = H // OH, W // OW
    nb = N // 2
    xt = x.transpose(2, 3, 0, 1)
    out_t = pl.pallas_call(
        functools.partial(_pool_kernel, OW, 1.0 / (sh * sw)),
        out_shape=jax.ShapeDtypeStruct((OH, OW, N, C), x.dtype),
        grid_spec=pl.GridSpec(
            grid=(N // nb, OH),
            in_specs=[
                pl.BlockSpec((sh, W, nb, C), lambda h, i: (i, 0, h, 0))
            ],
            out_specs=pl.BlockSpec((1, OW, nb, C), lambda h, i: (i, 0, h, 0)),
        ),
        compiler_params=pltpu.CompilerParams(
            dimension_semantics=("parallel", "parallel"),
        ),
        cost_estimate=pl.CostEstimate(
            flops=N * C * H * W,
            transcendentals=0,
            bytes_accessed=4 * N * C * (H * W + OH * OW),
        ),
    )(xt)
    return out_t.transpose(2, 3, 0, 1)
